# baseline (speedup 1.0000x reference)
import jax, jax.numpy as jnp
import numpy as np
from jax.experimental import pallas as pl

B = 1
SEQ = 2048
PRED = 2048
D = 768
H = 12
DFF = 1024
N_HASHES = 4
BUCKET = 4


def _pe(L, Dm):
    pos = np.arange(L, dtype=np.float32)[:, None]
    div = np.exp(np.arange(0, Dm, 2, dtype=np.float32) * (-np.log(10000.0) / Dm))
    pe = np.zeros((L, Dm), dtype=np.float32)
    pe[:, 0::2] = np.sin(pos * div)
    pe[:, 1::2] = np.cos(pos * div)
    return jnp.asarray(pe)


def _ln(x, g, b):
    m = jnp.mean(x, axis=-1, keepdims=True)
    v = jnp.var(x, axis=-1, keepdims=True)
    return (x - m) / jnp.sqrt(v + 1e-5) * g + b


def _token_conv(x, w):
    L = x.shape[1]
    xp = jnp.concatenate([x[:, -1:], x, x[:, :1]], axis=1)
    out = jnp.zeros((x.shape[0], L, w.shape[0]), jnp.float32)
    for k in range(3):
        out = out + jnp.einsum('blc,dc->bld', xp[:, k:k + L], w[:, :, k])
    return out


def _lsh_attention(x, Wqk, Wv, Wo, bo, key):
    Bb, L, Dd = x.shape
    Dh = Dd // H
    qk = (x @ Wqk).reshape(Bb, L, H, Dh).transpose(0, 2, 1, 3).reshape(Bb * H, L, Dh)
    v = (x @ Wv).reshape(Bb, L, H, Dh).transpose(0, 2, 1, 3).reshape(Bb * H, L, Dh)
    nb = L // BUCKET
    rot = jax.random.normal(key, (Dh, N_HASHES, nb // 2), jnp.float32)
    qk_sg = jax.lax.stop_gradient(qk)
    pos = jnp.arange(L)
    outs = []
    lgs = []
    nc = L // BUCKET
    for h in range(N_HASHES):
        rotated = jnp.einsum('bld,dr->blr', qk_sg, rot[:, h])
        bh = jnp.argmax(jnp.concatenate([rotated, -rotated], axis=-1), axis=-1)
        ticker = jnp.argsort(bh * L + pos[None, :], axis=-1)
        sqk = jnp.take_along_axis(qk, ticker[..., None], axis=1)
        sv = jnp.take_along_axis(v, ticker[..., None], axis=1)
        q = sqk.reshape(-1, nc, BUCKET, Dh)
        kk = sqk / (jnp.linalg.norm(sqk, axis=-1, keepdims=True) + 1e-9)
        kk = kk.reshape(-1, nc, BUCKET, Dh)
        vv = sv.reshape(-1, nc, BUCKET, Dh)
        spos = ticker.reshape(-1, nc, BUCKET)
        lob = lambda t: jnp.concatenate([jnp.roll(t, 1, axis=1), t], axis=2)
        bk = lob(kk)
        bv = lob(vv)
        bpos = lob(spos)
        dots = jnp.einsum('bnid,bnjd->bnij', q, bk) / (float(Dh) ** 0.5)
        selfm = spos[..., None] == bpos[:, :, None, :]
        dots = jnp.where(selfm, dots - 1e5, dots)
        lg = jax.nn.logsumexp(dots, axis=-1, keepdims=True)
        pr = jnp.exp(dots - lg)
        o = jnp.einsum('bnij,bnjd->bnid', pr, bv).reshape(-1, L, Dh)
        lgf = lg.reshape(-1, L)
        inv = jnp.argsort(ticker, axis=-1)
        outs.append(jnp.take_along_axis(o, inv[..., None], axis=1))
        lgs.append(jnp.take_along_axis(lgf, inv, axis=1))
    os_ = jnp.stack(outs, axis=0)
    lg_ = jnp.stack(lgs, axis=0)
    w = jax.nn.softmax(lg_, axis=0)[..., None]
    o = jnp.sum(os_ * w, axis=0)
    o = o.reshape(Bb, H, L, Dh).transpose(0, 2, 1, 3).reshape(Bb, L, Dd)
    return o @ Wo + bo


def _id_kernel(h_ref, o_ref):
    o_ref[...] = h_ref[...]


def _proj_kernel(h_ref, w_ref, o_ref):
    o_ref[...] = jax.lax.dot_general(
        h_ref[...], w_ref[...], (((1,), (0,)), ((), ())),
        precision=jax.lax.Precision.HIGHEST)


def kernel(x_enc, x_mark_enc, x_dec, x_mark_dec, params):
    x = jnp.concatenate([x_enc, x_dec[:, -PRED:, :]], axis=1)
    xm = jnp.concatenate([x_mark_enc, x_mark_dec[:, -PRED:, :]], axis=1)
    h = _token_conv(x, params['conv_w']) + xm @ params['temp_w'] + _pe(x.shape[1], D)[None]
    for li, lp in enumerate(params['layers']):
        key = jax.random.fold_in(jax.random.key(42), li)
        a = _lsh_attention(h, lp['Wqk'], lp['Wv'], lp['Wo'], lp['bo'], key)
        h = h + a
        h = _ln(h, lp['g1'], lp['b1'])
        y = jax.nn.gelu(h @ lp['c1w'] + lp['c1b'])
        y = y @ lp['c2w'] + lp['c2b']
        h = _ln(h + y, lp['g2'], lp['b2'])
    h = _ln(h, params['gn'], params['bn'])
    out = jnp.einsum('bld,dp->blp', h, params['Wp'],
                     precision=jax.lax.Precision.HIGHEST) + params['bp']
    out = out[:, -PRED:][..., None]
    out2 = pl.pallas_call(
        _id_kernel,
        out_shape=jax.ShapeDtypeStruct((PRED, 1), jnp.float32),
    )(out[0, :, 0, :])
    return out2[None, :, :, None]


# speedup vs baseline: 2.0158x; 2.0158x over previous
import functools
import jax
import jax.numpy as jnp
import numpy as np
from jax import lax
from jax.experimental import pallas as pl
from jax.experimental.pallas import tpu as pltpu
from jax.experimental.pallas import tpu_sc as plsc

B = 1
SEQ = 2048
PRED = 2048
D = 768
H = 12
DH = 64
DFF = 1024
LAYERS = 2
N_HASHES = 4
BUCKET = 4
L = SEQ + PRED
NB = L // BUCKET // 2
NT = H * N_HASHES
LE = L + 2 * BUCKET

_BF = jnp.bfloat16
_HI = jax.lax.Precision.HIGHEST


def _bfdot(a, b, dims):
    return jax.lax.dot_general(a.astype(_BF), b.astype(_BF), (dims, ((), ())),
                               preferred_element_type=jnp.float32)


def _pe_np(Lh, Dm):
    pos = np.arange(Lh, dtype=np.float32)[:, None]
    div = np.exp(np.arange(0, Dm, 2, dtype=np.float32) * (-np.log(10000.0) / Dm))
    pe = np.zeros((Lh, Dm), dtype=np.float32)
    pe[:, 0::2] = np.sin(pos * div)
    pe[:, 1::2] = np.cos(pos * div)
    return pe


def _embed_kernel(x32_ref, w32_ref, pe_ref, o_ref):
    x32 = x32_ref[...]
    w32 = w32_ref[...]
    e0 = _bfdot(x32[:, 0:8], w32[0:8], ((1,), (0,)))
    e1 = _bfdot(x32[:, 8:16], w32[8:16], ((1,), (0,)))
    e2 = _bfdot(x32[:, 16:24], w32[16:24], ((1,), (0,)))
    em = _bfdot(x32[:, 24:32], w32[24:32], ((1,), (0,)))
    o_ref[...] = (((e0 + e1) + e2) + em) + pe_ref[...]


def _embed(x, xm, conv_w, temp_w, pe):
    xp = jnp.concatenate([x[-1:], x, x[:1]], axis=0)
    z1 = jnp.zeros((L, 1), jnp.float32)
    z4 = jnp.zeros((L, 4), jnp.float32)
    taps = []
    for k in range(3):
        taps += [xp[k:k + L], z1]
    x32 = jnp.concatenate(taps + [xm, z4], axis=1)
    zw1 = jnp.zeros((1, D), jnp.float32)
    zw4 = jnp.zeros((4, D), jnp.float32)
    w_taps = []
    for k in range(3):
        w_taps += [conv_w[:, :, k].T, zw1]
    w32 = jnp.concatenate(w_taps + [temp_w, zw4], axis=0)
    return pl.pallas_call(
        _embed_kernel,
        out_shape=jax.ShapeDtypeStruct((L, D), jnp.float32),
        grid=(8,),
        in_specs=[pl.BlockSpec((L // 8, 32), lambda i: (i, 0)),
                  pl.BlockSpec((32, D), lambda i: (0, 0)),
                  pl.BlockSpec((L // 8, D), lambda i: (i, 0))],
        out_specs=pl.BlockSpec((L // 8, D), lambda i: (i, 0)),
    )(x32, w32, pe)


_LCH = 512


def _qkv_hash_kernel(h_ref, wqk_ref, wv_ref, rot_ref, qk_ref, v_ref, bh_ref):
    hh = h_ref[...]
    qk = _bfdot(hh, wqk_ref[0], ((1,), (0,)))
    vv = _bfdot(hh, wv_ref[0], ((1,), (0,)))
    qk_ref[0] = qk
    v_ref[0] = vv
    rotated = _bfdot(qk, rot_ref[...], ((1,), (0,)))
    jcol = jax.lax.broadcasted_iota(jnp.int32, (_LCH, NB), 1)
    for hsh in range(N_HASHES):
        r = rotated[:, hsh * NB:(hsh + 1) * NB]
        m1 = jnp.max(r, axis=1, keepdims=True)
        i1 = jnp.min(jnp.where(r == m1, jcol, L), axis=1)
        m2 = jnp.min(r, axis=1, keepdims=True)
        i2 = jnp.min(jnp.where(r == m2, jcol, L), axis=1)
        bh = jnp.where(m1[:, 0] >= -m2[:, 0], i1, NB + i2)
        bh_ref[hsh, :] = bh


def _qkv_hash(h, wqk, wv, rot2d):
    wqk_h = wqk.reshape(D, H, DH).transpose(1, 0, 2)
    wv_h = wv.reshape(D, H, DH).transpose(1, 0, 2)
    grid = (L // _LCH, H)
    return pl.pallas_call(
        _qkv_hash_kernel,
        out_shape=(jax.ShapeDtypeStruct((H, L, DH), jnp.float32),
                   jax.ShapeDtypeStruct((H, L, DH), jnp.float32),
                   jax.ShapeDtypeStruct((8 * H, L), jnp.int32)),
        grid=grid,
        in_specs=[pl.BlockSpec((_LCH, D), lambda i, hd: (i, 0)),
                  pl.BlockSpec((1, D, DH), lambda i, hd: (hd, 0, 0)),
                  pl.BlockSpec((1, D, DH), lambda i, hd: (hd, 0, 0)),
                  pl.BlockSpec((DH, N_HASHES * NB), lambda i, hd: (0, 0))],
        out_specs=(pl.BlockSpec((1, _LCH, DH), lambda i, hd: (hd, i, 0)),
                   pl.BlockSpec((1, _LCH, DH), lambda i, hd: (hd, i, 0)),
                   pl.BlockSpec((8, _LCH), lambda i, hd: (hd, i))),
    )(h, wqk_h, wv_h, rot2d)


_QT = 128
_KW = _QT + 2 * BUCKET


def _attn_kernel(sqk_ref, sv_ref, o_ref, kk_ref):
    sqk = sqk_ref[0]
    nrm = jnp.sqrt(jnp.sum(sqk * sqk, axis=1, keepdims=True))
    kk_ref[...] = sqk / (nrm + 1e-9)
    ii = jax.lax.broadcasted_iota(jnp.int32, (_QT, _KW), 0)
    jj = jax.lax.broadcasted_iota(jnp.int32, (_QT, _KW), 1)
    cd = (jj >> 2) - (ii >> 2)
    bandm = (cd == 1) | (cd == 2)
    selfm = jj == ii + 2 * BUCKET

    def tile(i, _):
        base = i * _QT
        q = sqk_ref[0, pl.ds(base + 2 * BUCKET, _QT), :]
        kw = kk_ref[pl.ds(base, _KW), :]
        dots = _bfdot(q, kw, ((1,), (1,))) / 8.0
        dots = jnp.where(bandm, jnp.where(selfm, dots - 1e5, dots), -1e30)
        m = jnp.max(dots, axis=1, keepdims=True)
        e = jnp.exp(dots - m)
        s = jnp.sum(e, axis=1, keepdims=True)
        lg = jnp.log(s) + m
        pr = jnp.exp(dots - lg)
        vw = sv_ref[0, pl.ds(base, _KW), :]
        o = _bfdot(pr, vw, ((1,), (0,)))
        o_ref[0, pl.ds(base, _QT), 0:DH] = o
        o_ref[0, pl.ds(base, _QT), DH:80] = jnp.broadcast_to(lg, (_QT, 16))
        return 0

    jax.lax.fori_loop(0, L // _QT, tile, 0)


def _attn(sqk_ext, sv_ext):
    return pl.pallas_call(
        _attn_kernel,
        out_shape=jax.ShapeDtypeStruct((NT, L, 80), jnp.float32),
        grid=(NT,),
        in_specs=[pl.BlockSpec((1, LE, DH), lambda t: (t, 0, 0)),
                  pl.BlockSpec((1, LE, DH), lambda t: (t, 0, 0))],
        out_specs=pl.BlockSpec((1, L, 80), lambda t: (t, 0, 0)),
        scratch_shapes=[pltpu.VMEM((LE, DH), jnp.float32)],
    )(sqk_ext, sv_ext)


_CT = 512


def _ln_rows(x, g, b):
    m = jnp.mean(x, axis=-1, keepdims=True)
    d = x - m
    v = jnp.mean(d * d, axis=-1, keepdims=True)
    return d / jnp.sqrt(v + 1e-5) * g + b


def _combine_kernel(o80_ref, h_ref, wo_ref, bo_ref, g1_ref, b1_ref,
                    c1w_ref, c1b_ref, c2w_ref, c2b_ref, g2_ref, b2_ref,
                    out_ref):
    heads = []
    for hd in range(H):
        lgs = [o80_ref[hd * N_HASHES + hs, :, DH:DH + 1] for hs in range(N_HASHES)]
        lg = jnp.concatenate(lgs, axis=1)
        mx = jnp.max(lg, axis=1, keepdims=True)
        ew = jnp.exp(lg - mx)
        sw = jnp.sum(ew, axis=1, keepdims=True)
        acc = jnp.zeros((_CT, DH), jnp.float32)
        for hs in range(N_HASHES):
            w = (ew[:, hs] / sw[:, 0])[:, None]
            acc = acc + o80_ref[hd * N_HASHES + hs, :, 0:DH] * w
        heads.append(acc)
    ocat = jnp.concatenate(heads, axis=1)
    a = _bfdot(ocat, wo_ref[...], ((1,), (0,))) + bo_ref[...]
    h1 = h_ref[...] + a
    h1n = _ln_rows(h1, g1_ref[...], b1_ref[...])
    y = jax.nn.gelu(_bfdot(h1n, c1w_ref[...], ((1,), (0,))) + c1b_ref[...])
    y2 = _bfdot(y, c2w_ref[...], ((1,), (0,))) + c2b_ref[...]
    out_ref[...] = _ln_rows(h1n + y2, g2_ref[...], b2_ref[...])


def _combine(o80, h, lp):
    grid = (L // _CT,)
    full = lambda *s: pl.BlockSpec(s, lambda i: tuple(0 for _ in s))
    return pl.pallas_call(
        _combine_kernel,
        out_shape=jax.ShapeDtypeStruct((L, D), jnp.float32),
        grid=grid,
        in_specs=[pl.BlockSpec((NT, _CT, 80), lambda i: (0, i, 0)),
                  pl.BlockSpec((_CT, D), lambda i: (i, 0)),
                  full(D, D), full(D), full(D), full(D),
                  full(D, DFF), full(DFF), full(DFF, D), full(D),
                  full(D), full(D)],
        out_specs=pl.BlockSpec((_CT, D), lambda i: (i, 0)),
    )(o80, h, lp['Wo'], lp['bo'], lp['g1'], lp['b1'],
      lp['c1w'], lp['c1b'], lp['c2w'], lp['c2b'], lp['g2'], lp['b2'])


def _final_kernel(h_ref, g_ref, b_ref, w_ref, o_ref):
    hn = _ln_rows(h_ref[...], g_ref[...], b_ref[...])
    o_ref[...] = _bfdot(hn, w_ref[...], ((1,), (0,)))


def _final(h, gn, bn, wp):
    wp128 = jnp.concatenate([wp, jnp.zeros((D, 127), jnp.float32)], axis=1)
    out = pl.pallas_call(
        _final_kernel,
        out_shape=jax.ShapeDtypeStruct((PRED, 128), jnp.float32),
        grid=(PRED // 512,),
        in_specs=[pl.BlockSpec((512, D), lambda i: (i + (L - PRED) // 512, 0)),
                  pl.BlockSpec((D,), lambda i: (0,)),
                  pl.BlockSpec((D,), lambda i: (0,)),
                  pl.BlockSpec((D, 128), lambda i: (0, 0))],
        out_specs=pl.BlockSpec((512, 128), lambda i: (i, 0)),
    )(h, gn, bn, wp128)
    return out[:, :1]


_GC = 256
_UC = 512


def _sc_sort_gather_kernel(bh96_ref, qkf_ref, vf_ref,
                           sqk_ref, sv_ref, invb_ref,
                           bh_t, inv_t, hist, wrapb,
                           bufq0, bufv0, semq0, semv0,
                           *scat_bufs):
    wid = lax.axis_index("s") * 2 + lax.axis_index("c")
    zeros16 = jnp.zeros((16,), jnp.int32)

    lanes16 = jax.lax.iota(jnp.int32, 16)

    def do_task(t):
        head = t // N_HASHES
        row96 = head * 8 + (t % N_HASHES)
        pltpu.sync_copy(bh96_ref.at[row96], bh_t)

        def zinit(j, _):
            hist[j] = 0
            return 0
        lax.fori_loop(0, 1024, zinit, 0)

        def phase_a(i, _):
            bv = bh_t[pl.ds(i * 16, 16)]
            for u in range(16):
                b = bv[u]
                hist[b] = hist[b] + 1
            return 0
        lax.fori_loop(0, L // 16, phase_a, 0)

        def phase_b(j, carry):
            for u in range(4):
                hv = hist[j * 4 + u]
                hist[j * 4 + u] = carry
                carry = carry + hv
            return carry
        lax.fori_loop(0, 256, phase_b, jnp.int32(0))

        for c in range(L // _GC):
            def phase_c(i, _, c=c):
                bv = bh_t[pl.ds(c * _GC + i * 16, 16)]
                racc = zeros16
                for u in range(16):
                    b = bv[u]
                    r = hist[b]
                    hist[b] = r + 1
                    racc = jnp.where(lanes16 == u, r, racc)
                inv_t[pl.ds(c * _GC + i * 16, 16)] = racc + t * L
                scat_bufs[c][pl.ds(i * 16, 16)] = (
                    racc + (t * LE + 2 * BUCKET))
                return 0
            lax.fori_loop(0, _GC // 16, phase_c, 0)

        pltpu.sync_copy(inv_t, invb_ref.at[t])

        nch = L // _GC
        for c in range(nch):
            idx_sl = scat_bufs[c]
            cq = pltpu.async_copy(
                qkf_ref.at[pl.ds(head * L + c * _GC, _GC)], bufq0, semq0)
            cv = pltpu.async_copy(
                vf_ref.at[pl.ds(head * L + c * _GC, _GC)], bufv0, semv0)
            cq.wait()
            pltpu.sync_copy(bufq0, sqk_ref.at[idx_sl])
            cv.wait()
            pltpu.sync_copy(bufv0, sv_ref.at[idx_sl])
        pltpu.sync_copy(sqk_ref.at[pl.ds(t * LE + L, 2 * BUCKET)], wrapb)
        pltpu.sync_copy(wrapb, sqk_ref.at[pl.ds(t * LE, 2 * BUCKET)])
        pltpu.sync_copy(sv_ref.at[pl.ds(t * LE + L, 2 * BUCKET)], wrapb)
        pltpu.sync_copy(wrapb, sv_ref.at[pl.ds(t * LE, 2 * BUCKET)])

    for r in range(2):
        t = wid + r * 32
        @pl.when(t < NT)
        def _():
            do_task(t)


def _sc_sort_gather(bh96, qkf, vf):
    kfn = pl.kernel(
        _sc_sort_gather_kernel,
        out_type=(jax.ShapeDtypeStruct((NT * LE, 1, DH), jnp.float32),
                  jax.ShapeDtypeStruct((NT * LE, 1, DH), jnp.float32),
                  jax.ShapeDtypeStruct((NT, L), jnp.int32)),
        mesh=plsc.VectorSubcoreMesh(core_axis_name="c", subcore_axis_name="s"),
        scratch_types=[
            pltpu.VMEM((L,), jnp.int32),
            pltpu.VMEM((L,), jnp.int32),
            pltpu.SMEM((1024,), jnp.int32),
            pltpu.VMEM((2 * BUCKET, 1, DH), jnp.float32),
            pltpu.VMEM((_GC, 1, DH), jnp.float32),
            pltpu.VMEM((_GC, 1, DH), jnp.float32),
            pltpu.SemaphoreType.DMA,
            pltpu.SemaphoreType.DMA,
        ] + [pltpu.VMEM((_GC,), jnp.int32) for _ in range(L // _GC)],
    )
    return kfn(bh96, qkf, vf)


def _sc_unsort_kernel(o80f_ref, invb_ref, out_ref,
                      buf0, sem0, *invbufs):
    wid = lax.axis_index("s") * 2 + lax.axis_index("c")

    def do_task(t):
        nch = L // _UC
        for c in range(nch):
            pltpu.sync_copy(invb_ref.at[t, pl.ds(c * _UC, _UC)], invbufs[c])
        for c in range(nch):
            cp = pltpu.async_copy(o80f_ref.at[invbufs[c]], buf0, sem0)
            cp.wait()
            pltpu.sync_copy(buf0, out_ref.at[t, pl.ds(c * _UC, _UC)])

    for r in range(2):
        t = wid + r * 32
        @pl.when(t < NT)
        def _():
            do_task(t)


def _sc_unsort(o80f, invb):
    kfn = pl.kernel(
        _sc_unsort_kernel,
        out_type=jax.ShapeDtypeStruct((NT, L, 1, 80), jnp.float32),
        mesh=plsc.VectorSubcoreMesh(core_axis_name="c", subcore_axis_name="s"),
        scratch_types=[
            pltpu.VMEM((_UC, 1, 80), jnp.float32),
            pltpu.SemaphoreType.DMA,
        ] + [pltpu.VMEM((_UC,), jnp.int32) for _ in range(L // _UC)],
    )
    return kfn(o80f, invb)


def _sort_gather_jnp(bh96, qk, v):
    pos = jnp.arange(L, dtype=jnp.int32)
    sqk_l, sv_l, inv_l = [], [], []
    for t in range(NT):
        hd, hs = t // N_HASHES, t % N_HASHES
        bh = bh96[hd * 8 + hs]
        ticker = jnp.argsort(bh * L + pos)
        inv = jnp.argsort(ticker).astype(jnp.int32)
        tick_ext = jnp.concatenate([ticker[-2 * BUCKET:], ticker])
        sqk_l.append(qk[hd][tick_ext])
        sv_l.append(v[hd][tick_ext])
        inv_l.append(inv + t * L)
    return (jnp.stack(sqk_l), jnp.stack(sv_l), jnp.stack(inv_l))


def _unsort_jnp(o80_sorted, inv_b):
    flat = o80_sorted.reshape(NT * L, 80)
    return flat[inv_b.reshape(-1)].reshape(NT, L, 80)


def _ref_lsh(x, Wqk, Wv, Wo, bo, key):
    Bb, Lx, Dd = x.shape
    Dh = Dd // H
    qk = (x @ Wqk).reshape(Bb, Lx, H, Dh).transpose(0, 2, 1, 3).reshape(Bb * H, Lx, Dh)
    v = (x @ Wv).reshape(Bb, Lx, H, Dh).transpose(0, 2, 1, 3).reshape(Bb * H, Lx, Dh)
    nb = Lx // BUCKET
    rot = jax.random.normal(key, (Dh, N_HASHES, nb // 2), jnp.float32)
    qk_sg = jax.lax.stop_gradient(qk)
    pos = jnp.arange(Lx)
    outs, lgs = [], []
    nc = Lx // BUCKET
    for hh in range(N_HASHES):
        rotated = jnp.einsum('bld,dr->blr', qk_sg, rot[:, hh])
        bh = jnp.argmax(jnp.concatenate([rotated, -rotated], axis=-1), axis=-1)
        ticker = jnp.argsort(bh * Lx + pos[None, :], axis=-1)
        sqk = jnp.take_along_axis(qk, ticker[..., None], axis=1)
        sv = jnp.take_along_axis(v, ticker[..., None], axis=1)
        q = sqk.reshape(-1, nc, BUCKET, Dh)
        kk = sqk / (jnp.linalg.norm(sqk, axis=-1, keepdims=True) + 1e-9)
        kk = kk.reshape(-1, nc, BUCKET, Dh)
        vv = sv.reshape(-1, nc, BUCKET, Dh)
        spos = ticker.reshape(-1, nc, BUCKET)
        lob = lambda t: jnp.concatenate([jnp.roll(t, 1, axis=1), t], axis=2)
        bk, bv, bpos = lob(kk), lob(vv), lob(spos)
        dots = jnp.einsum('bnid,bnjd->bnij', q, bk) / (float(Dh) ** 0.5)
        selfm = spos[..., None] == bpos[:, :, None, :]
        dots = jnp.where(selfm, dots - 1e5, dots)
        lg = jax.nn.logsumexp(dots, axis=-1, keepdims=True)
        pr = jnp.exp(dots - lg)
        o = jnp.einsum('bnij,bnjd->bnid', pr, bv).reshape(-1, Lx, Dh)
        lgf = lg.reshape(-1, Lx)
        inv = jnp.argsort(ticker, axis=-1)
        outs.append(jnp.take_along_axis(o, inv[..., None], axis=1))
        lgs.append(jnp.take_along_axis(lgf, inv, axis=1))
    os_ = jnp.stack(outs, axis=0)
    lg_ = jnp.stack(lgs, axis=0)
    w = jax.nn.softmax(lg_, axis=0)[..., None]
    o = jnp.sum(os_ * w, axis=0)
    o = o.reshape(Bb, H, Lx, Dh).transpose(0, 2, 1, 3).reshape(Bb, Lx, Dd)
    return o @ Wo + bo


def _attn_rest_jnp(qk, v, bh96, lp):
    pos = jnp.arange(L)
    outs, lgs = [], []
    nc = L // BUCKET
    for hs in range(N_HASHES):
        bh = jnp.stack([bh96[hd * 8 + hs] for hd in range(H)])
        ticker = jnp.argsort(bh * L + pos[None, :], axis=-1)
        sqk = jnp.take_along_axis(qk, ticker[..., None], axis=1)
        sv = jnp.take_along_axis(v, ticker[..., None], axis=1)
        q = sqk.reshape(-1, nc, BUCKET, DH)
        kk = sqk / (jnp.linalg.norm(sqk, axis=-1, keepdims=True) + 1e-9)
        kk = kk.reshape(-1, nc, BUCKET, DH)
        vv = sv.reshape(-1, nc, BUCKET, DH)
        spos = ticker.reshape(-1, nc, BUCKET)
        lob = lambda t: jnp.concatenate([jnp.roll(t, 1, axis=1), t], axis=2)
        bk, bv, bpos = lob(kk), lob(vv), lob(spos)
        dots = jnp.einsum('bnid,bnjd->bnij', q, bk) / (float(DH) ** 0.5)
        selfm = spos[..., None] == bpos[:, :, None, :]
        dots = jnp.where(selfm, dots - 1e5, dots)
        lg = jax.nn.logsumexp(dots, axis=-1, keepdims=True)
        pr = jnp.exp(dots - lg)
        o = jnp.einsum('bnij,bnjd->bnid', pr, bv).reshape(-1, L, DH)
        lgf = lg.reshape(-1, L)
        inv = jnp.argsort(ticker, axis=-1)
        outs.append(jnp.take_along_axis(o, inv[..., None], axis=1))
        lgs.append(jnp.take_along_axis(lgf, inv, axis=1))
    os_ = jnp.stack(outs, axis=0)
    lg_ = jnp.stack(lgs, axis=0)
    w = jax.nn.softmax(lg_, axis=0)[..., None]
    o = jnp.sum(os_ * w, axis=0)
    o = o.reshape(1, H, L, DH).transpose(0, 2, 1, 3).reshape(1, L, D)
    return (o @ lp['Wo'] + lp['bo'])[0]


def _rest_jnp(h, a, lp):
    def ln(x, g, b):
        m = jnp.mean(x, axis=-1, keepdims=True)
        vv = jnp.var(x, axis=-1, keepdims=True)
        return (x - m) / jnp.sqrt(vv + 1e-5) * g + b
    h = ln(h + a, lp['g1'], lp['b1'])
    y = jax.nn.gelu(h @ lp['c1w'] + lp['c1b'])
    y = y @ lp['c2w'] + lp['c2b']
    return ln(h + y, lp['g2'], lp['b2'])


def kernel(x_enc, x_mark_enc, x_dec, x_mark_dec, params):
    x = jnp.concatenate([x_enc, x_dec[:, -PRED:, :]], axis=1)[0]
    xm = jnp.concatenate([x_mark_enc, x_mark_dec[:, -PRED:, :]], axis=1)[0]
    pe = jnp.asarray(_pe_np(L, D))
    h = _embed(x, xm, params['conv_w'], params['temp_w'], pe)
    for li, lp in enumerate(params['layers']):
        key = jax.random.fold_in(jax.random.key(42), li)
        rot = jax.random.normal(key, (DH, N_HASHES, NB), jnp.float32)
        rot2d = rot.reshape(DH, N_HASHES * NB)
        qk, v, bh96 = _qkv_hash(h, lp['Wqk'], lp['Wv'], rot2d)
        sqk_ext, sv_ext, inv_b = _sort_gather_jnp(bh96, qk, v)
        o80s = _attn(sqk_ext.reshape(NT, LE, DH), sv_ext.reshape(NT, LE, DH))
        o80 = _unsort_jnp(o80s, inv_b)
        h = _combine(o80, h, lp)
    out = _final(h, params['gn'], params['bn'], params['Wp'])
    return (out + params['bp'])[None, :, :, None]
